# Initial kernel scaffold; baseline (speedup 1.0000x reference)
import functools

import jax
import jax.numpy as jnp
from jax.experimental import pallas as pl
from jax.experimental.pallas import tpu as pltpu


def _moe_dense_body(x_ref, sel_w_ref, keys_ref, values_ref, out_ref, g_ref):
    e = pl.program_id(1)
    E = sel_w_ref.shape[1]
    Bt = x_ref.shape[0]

    @pl.when(e == 0)
    def _router():
        logits = jnp.dot(x_ref[...], sel_w_ref[...],
                         preferred_element_type=jnp.float32)
        s = jax.nn.sigmoid(logits)
        iota_e = jax.lax.broadcasted_iota(jnp.int32, (Bt, E), 1)
        m1 = jnp.max(s, axis=1, keepdims=True)
        i1 = jnp.min(jnp.where(s == m1, iota_e, E), axis=1, keepdims=True)
        s2 = jnp.where(iota_e == i1, -jnp.inf, s)
        m2 = jnp.max(s2, axis=1, keepdims=True)
        i2 = jnp.min(jnp.where(s2 == m2, iota_e, E), axis=1, keepdims=True)
        g_ref[...] = jnp.where((iota_e == i1) | (iota_e == i2), s, 0.0)

    h = jnp.dot(x_ref[...], keys_ref[0], preferred_element_type=jnp.float32)
    h = jnp.maximum(h, 0.0)
    o = jnp.dot(h, values_ref[0], preferred_element_type=jnp.float32)
    contrib = o * g_ref[:, pl.ds(e, 1)]

    @pl.when(e == 0)
    def _init():
        out_ref[...] = contrib

    @pl.when(e != 0)
    def _acc():
        out_ref[...] += contrib


def kernel(x, expert_sel, keys_w, values_w):
    B, S, D = x.shape
    E = expert_sel.shape[1]
    F = keys_w.shape[2]
    T = B * S
    tokens = x.reshape(T, D)

    Bt = 512 if T % 512 == 0 else T
    grid = (T // Bt, E)

    out = pl.pallas_call(
        _moe_dense_body,
        grid=grid,
        in_specs=[
            pl.BlockSpec((Bt, D), lambda t, e: (t, 0)),
            pl.BlockSpec((D, E), lambda t, e: (0, 0)),
            pl.BlockSpec((1, D, F), lambda t, e: (e, 0, 0)),
            pl.BlockSpec((1, F, D), lambda t, e: (e, 0, 0)),
        ],
        out_specs=pl.BlockSpec((Bt, D), lambda t, e: (t, 0)),
        out_shape=jax.ShapeDtypeStruct((T, D), jnp.float32),
        scratch_shapes=[pltpu.VMEM((Bt, E), jnp.float32)],
    )(tokens, expert_sel, keys_w, values_w)

    return out.reshape(B, S, D)


# speedup vs baseline: 53.5752x; 53.5752x over previous
import functools

import jax
import jax.numpy as jnp
from jax.experimental import pallas as pl
from jax.experimental.pallas import tpu as pltpu


def _moe_dense_body(x_ref, sel_w_ref, keys_ref, values_ref, out_ref, g_ref):
    e = pl.program_id(1)
    E = sel_w_ref.shape[1]
    Bt = x_ref.shape[0]

    @pl.when(e == 0)
    def _router():
        logits = jnp.dot(x_ref[...], sel_w_ref[...],
                         preferred_element_type=jnp.float32)
        s = jax.nn.sigmoid(logits)
        iota_e = jax.lax.broadcasted_iota(jnp.int32, (Bt, E), 1)
        m1 = jnp.max(s, axis=1, keepdims=True)
        i1 = jnp.min(jnp.where(s == m1, iota_e, E), axis=1, keepdims=True)
        s2 = jnp.where(iota_e == i1, -jnp.inf, s)
        m2 = jnp.max(s2, axis=1, keepdims=True)
        i2 = jnp.min(jnp.where(s2 == m2, iota_e, E), axis=1, keepdims=True)
        g_ref[...] = jnp.where((iota_e == i1) | (iota_e == i2), s, 0.0)

    h = jnp.dot(x_ref[...], keys_ref[0], preferred_element_type=jnp.float32)
    h = jnp.maximum(h, 0.0)
    o = jnp.dot(h, values_ref[0], preferred_element_type=jnp.float32)
    iota_g = jax.lax.broadcasted_iota(jnp.int32, (Bt, E), 1)
    g_col = jnp.sum(jnp.where(iota_g == e, g_ref[...], 0.0), axis=1,
                    keepdims=True)
    contrib = o * g_col

    @pl.when(e == 0)
    def _init():
        out_ref[...] = contrib

    @pl.when(e != 0)
    def _acc():
        out_ref[...] += contrib


def kernel(x, expert_sel, keys_w, values_w):
    B, S, D = x.shape
    E = expert_sel.shape[1]
    F = keys_w.shape[2]
    T = B * S
    tokens = x.reshape(T, D)

    Bt = 512 if T % 512 == 0 else T
    grid = (T // Bt, E)

    out = pl.pallas_call(
        _moe_dense_body,
        grid=grid,
        in_specs=[
            pl.BlockSpec((Bt, D), lambda t, e: (t, 0)),
            pl.BlockSpec((D, E), lambda t, e: (0, 0)),
            pl.BlockSpec((1, D, F), lambda t, e: (e, 0, 0)),
            pl.BlockSpec((1, F, D), lambda t, e: (e, 0, 0)),
        ],
        out_specs=pl.BlockSpec((Bt, D), lambda t, e: (t, 0)),
        out_shape=jax.ShapeDtypeStruct((T, D), jnp.float32),
        scratch_shapes=[pltpu.VMEM((Bt, E), jnp.float32)],
    )(tokens, expert_sel, keys_w, values_w)

    return out.reshape(B, S, D)


# speedup vs baseline: 53.6410x; 1.0012x over previous
import functools

import jax
import jax.numpy as jnp
from jax.experimental import pallas as pl
from jax.experimental.pallas import tpu as pltpu


def _moe_dense_body(x_ref, sel_w_ref, keys_ref, values_ref, out_ref, g_ref):
    e = pl.program_id(1)
    E = sel_w_ref.shape[1]
    Bt = x_ref.shape[0]

    @pl.when(e == 0)
    def _router():
        logits = jnp.dot(x_ref[...], sel_w_ref[...],
                         preferred_element_type=jnp.float32)
        s = jax.nn.sigmoid(logits)
        iota_e = jax.lax.broadcasted_iota(jnp.int32, (Bt, E), 1)
        m1 = jnp.max(s, axis=1, keepdims=True)
        i1 = jnp.min(jnp.where(s == m1, iota_e, E), axis=1, keepdims=True)
        s2 = jnp.where(iota_e == i1, -jnp.inf, s)
        m2 = jnp.max(s2, axis=1, keepdims=True)
        i2 = jnp.min(jnp.where(s2 == m2, iota_e, E), axis=1, keepdims=True)
        g_ref[...] = jnp.where((iota_e == i1) | (iota_e == i2), s, 0.0)

    xb = x_ref[...].astype(jnp.bfloat16)
    h = jnp.dot(xb, keys_ref[0].astype(jnp.bfloat16),
                preferred_element_type=jnp.float32)
    h = jnp.maximum(h, 0.0)
    o = jnp.dot(h.astype(jnp.bfloat16), values_ref[0].astype(jnp.bfloat16),
                preferred_element_type=jnp.float32)
    iota_g = jax.lax.broadcasted_iota(jnp.int32, (Bt, E), 1)
    g_col = jnp.sum(jnp.where(iota_g == e, g_ref[...], 0.0), axis=1,
                    keepdims=True)
    contrib = o * g_col

    @pl.when(e == 0)
    def _init():
        out_ref[...] = contrib

    @pl.when(e != 0)
    def _acc():
        out_ref[...] += contrib


def kernel(x, expert_sel, keys_w, values_w):
    B, S, D = x.shape
    E = expert_sel.shape[1]
    F = keys_w.shape[2]
    T = B * S
    tokens = x.reshape(T, D)

    Bt = 512 if T % 512 == 0 else T
    grid = (T // Bt, E)

    out = pl.pallas_call(
        _moe_dense_body,
        grid=grid,
        in_specs=[
            pl.BlockSpec((Bt, D), lambda t, e: (t, 0)),
            pl.BlockSpec((D, E), lambda t, e: (0, 0)),
            pl.BlockSpec((1, D, F), lambda t, e: (e, 0, 0)),
            pl.BlockSpec((1, F, D), lambda t, e: (e, 0, 0)),
        ],
        out_specs=pl.BlockSpec((Bt, D), lambda t, e: (t, 0)),
        out_shape=jax.ShapeDtypeStruct((T, D), jnp.float32),
        scratch_shapes=[pltpu.VMEM((Bt, E), jnp.float32)],
    )(tokens, expert_sel, keys_w, values_w)

    return out.reshape(B, S, D)


# speedup vs baseline: 58.9990x; 1.0999x over previous
import functools

import jax
import jax.numpy as jnp
from jax.experimental import pallas as pl
from jax.experimental.pallas import tpu as pltpu


def _moe_dense_body(x_ref, sel_w_ref, keys_ref, values_ref, out_ref, g_ref):
    e = pl.program_id(1)
    E = sel_w_ref.shape[1]
    Bt = x_ref.shape[0]

    @pl.when(e == 0)
    def _router():
        logits = jnp.dot(x_ref[...], sel_w_ref[...],
                         preferred_element_type=jnp.float32)
        s = jax.nn.sigmoid(logits)
        iota_e = jax.lax.broadcasted_iota(jnp.int32, (Bt, E), 1)
        m1 = jnp.max(s, axis=1, keepdims=True)
        i1 = jnp.min(jnp.where(s == m1, iota_e, E), axis=1, keepdims=True)
        s2 = jnp.where(iota_e == i1, -jnp.inf, s)
        m2 = jnp.max(s2, axis=1, keepdims=True)
        i2 = jnp.min(jnp.where(s2 == m2, iota_e, E), axis=1, keepdims=True)
        g_ref[...] = jnp.where((iota_e == i1) | (iota_e == i2), s, 0.0)

    xb = x_ref[...].astype(jnp.bfloat16)
    h = jnp.dot(xb, keys_ref[0], preferred_element_type=jnp.float32)
    h = jnp.maximum(h, 0.0)
    o = jnp.dot(h.astype(jnp.bfloat16), values_ref[0],
                preferred_element_type=jnp.float32)
    iota_g = jax.lax.broadcasted_iota(jnp.int32, (Bt, E), 1)
    g_col = jnp.sum(jnp.where(iota_g == e, g_ref[...], 0.0), axis=1,
                    keepdims=True)
    contrib = o * g_col

    @pl.when(e == 0)
    def _init():
        out_ref[...] = contrib

    @pl.when(e != 0)
    def _acc():
        out_ref[...] += contrib


def kernel(x, expert_sel, keys_w, values_w):
    B, S, D = x.shape
    E = expert_sel.shape[1]
    F = keys_w.shape[2]
    T = B * S
    tokens = x.reshape(T, D)

    Bt = 1024 if T % 1024 == 0 else T
    grid = (T // Bt, E)

    out = pl.pallas_call(
        _moe_dense_body,
        grid=grid,
        in_specs=[
            pl.BlockSpec((Bt, D), lambda t, e: (t, 0)),
            pl.BlockSpec((D, E), lambda t, e: (0, 0)),
            pl.BlockSpec((1, D, F), lambda t, e: (e, 0, 0)),
            pl.BlockSpec((1, F, D), lambda t, e: (e, 0, 0)),
        ],
        out_specs=pl.BlockSpec((Bt, D), lambda t, e: (t, 0)),
        out_shape=jax.ShapeDtypeStruct((T, D), jnp.float32),
        scratch_shapes=[pltpu.VMEM((Bt, E), jnp.float32)],
    )(tokens, expert_sel, keys_w.astype(jnp.bfloat16),
      values_w.astype(jnp.bfloat16))

    return out.reshape(B, S, D)


# speedup vs baseline: 85.1377x; 1.4430x over previous
import functools

import jax
import jax.numpy as jnp
from jax import lax
from jax.experimental import pallas as pl
from jax.experimental.pallas import tpu as pltpu
from jax.experimental.pallas import tpu_sc as plsc

BM = 256
NW = 32
RG = 16


def _take16(v, idx):
    dnums = lax.GatherDimensionNumbers(
        offset_dims=(), collapsed_slice_dims=(0,), start_index_map=(0,))
    return lax.gather(v, idx[:, None], dnums, slice_sizes=(1,),
                      mode=lax.GatherScatterMode.PROMISE_IN_BOUNDS)


def _splat(v, lane):
    return _take16(v, jnp.full((16,), lane, jnp.int32))


def _prefix_incl(s, iota16):
    for k in (1, 2, 4, 8):
        g = _take16(s, jnp.maximum(iota16 - k, 0))
        s = s + jnp.where(iota16 >= k, g, jnp.zeros_like(s))
    return s


def _router_body(x_ref, sel_w_ref, idx_ref, gv_ref, hist_ref):
    Bt = x_ref.shape[0]
    E = sel_w_ref.shape[1]
    logits = jnp.dot(x_ref[...], sel_w_ref[...],
                     preferred_element_type=jnp.float32)
    s = jax.nn.sigmoid(logits)
    iota_e = lax.broadcasted_iota(jnp.int32, (Bt, E), 1)
    m1 = jnp.max(s, axis=1, keepdims=True)
    i1 = jnp.min(jnp.where(s == m1, iota_e, E), axis=1, keepdims=True)
    s2 = jnp.where(iota_e == i1, -jnp.inf, s)
    m2 = jnp.max(s2, axis=1, keepdims=True)
    i2 = jnp.min(jnp.where(s2 == m2, iota_e, E), axis=1, keepdims=True)
    idx_ref[...] = jnp.concatenate([i1, i2], axis=1)
    gv_ref[...] = jnp.concatenate([m1, m2], axis=1)
    half = Bt // 2
    rows = []
    for h in range(2):
        i1h = i1[h * half:(h + 1) * half]
        i2h = i2[h * half:(h + 1) * half]
        ioh = lax.broadcasted_iota(jnp.int32, (half, E), 1)
        cnt = ((i1h == ioh).astype(jnp.int32)
               + (i2h == ioh).astype(jnp.int32))
        rows.append(jnp.sum(cnt, axis=0, keepdims=True))
    hist_ref[...] = jnp.concatenate(rows, axis=0)[None]


def _dispatch_body(idx_hbm, x_hbm, hist_hbm, xs_hbm, pos_hbm, be_hbm,
                   idxv, histv, destv, tok16, dest16, tokv, bev, rowbuf,
                   sem1, sem2):
    T, D = x_hbm.shape
    NP = idx_hbm.shape[0]
    E = 16
    CH = NP // NW
    NB = be_hbm.shape[0]
    wid = lax.axis_index("s") * 2 + lax.axis_index("c")
    base_p = pl.multiple_of(wid * CH, 8)
    iota16 = lax.iota(jnp.int32, 16)
    ones16 = jnp.full((16,), 1, jnp.int32)
    zeros16 = jnp.zeros((16,), jnp.int32)

    pltpu.sync_copy(idx_hbm.at[pl.ds(base_p, CH)], idxv)
    pltpu.sync_copy(hist_hbm, histv)

    def h_body(c, carry):
        ctot, pre = carry
        row = histv[pl.ds(c * E, E)]
        ctot = ctot + row
        pre = pre + jnp.where(c < wid, row, zeros16)
        return ctot, pre

    ctot, pre = lax.fori_loop(0, NW, h_body, (zeros16, zeros16))
    pc = lax.shift_left(lax.shift_right_logical(ctot + (BM - 1), 8), 8)
    gs = _prefix_incl(pc, iota16) - pc
    base = gs + pre

    for q in range(NB // 16):
        bstart = (iota16 + q * 16) * BM
        be = jnp.full((16,), -1, jnp.int32)
        for e in range(E):
            gse = _splat(gs, e)
            pce = _splat(pc, e)
            m = (bstart >= gse) & (bstart < gse + pce)
            be = jnp.where(m, e, be)
        bev[pl.ds(q * 16, 16)] = be

    @pl.when(wid == 0)
    def _():
        pltpu.sync_copy(bev, be_hbm)

    def c_body(j, cnt):
        v = idxv[pl.ds(j * 16, 16)]
        dest = zeros16
        for e in range(E):
            m = v == e
            mi = jnp.where(m, ones16, zeros16)
            pref = _prefix_incl(mi, iota16)
            ce = _splat(cnt, e)
            nm = _splat(pref, 15)
            dest = jnp.where(m, ce + pref - 1, dest)
            cnt = cnt + jnp.where(iota16 == e, nm, zeros16)
        destv[pl.ds(j * 16, 16)] = dest
        p = base_p + j * 16 + iota16
        tokv[pl.ds(j * 16, 16)] = lax.shift_right_logical(p, 1)
        return cnt

    lax.fori_loop(0, CH // 16, c_body, base)
    pltpu.sync_copy(destv, pos_hbm.at[pl.ds(base_p, CH)])

    def d_body(j, _):
        tok16[pl.ds(0, RG)] = tokv[pl.ds(j * RG, RG)]
        dest16[pl.ds(0, RG)] = destv[pl.ds(j * RG, RG)]
        pltpu.async_copy(x_hbm.at[tok16], rowbuf, sem1).wait()
        pltpu.async_copy(rowbuf, xs_hbm.at[dest16], sem2).wait()
        return 0

    lax.fori_loop(0, CH // RG, d_body, 0)


def _gmm_body(be_ref, xs_ref, k_ref, v_ref, os_ref):
    xb = xs_ref[...].astype(jnp.bfloat16)
    h = jnp.dot(xb, k_ref[0], preferred_element_type=jnp.float32)
    h = jnp.maximum(h, 0.0)
    o = jnp.dot(h.astype(jnp.bfloat16), v_ref[0],
                preferred_element_type=jnp.float32)
    os_ref[...] = o


def _combine_body(os_hbm, pos_hbm, gv_hbm, out_hbm,
                  posv, gvv, p16, rbuf, obuf, semA):
    D = os_hbm.shape[1]
    NP = pos_hbm.shape[0]
    CH = NP // NW
    wid = lax.axis_index("s") * 2 + lax.axis_index("c")
    base_p = pl.multiple_of(wid * CH, 8)
    base_t = pl.multiple_of(wid * (CH // 2), 8)

    pltpu.sync_copy(pos_hbm.at[pl.ds(base_p, CH)], posv)
    pltpu.sync_copy(gv_hbm.at[pl.ds(base_p, CH)], gvv)

    def c_body(c, _):
        p16[pl.ds(0, 16)] = posv[pl.ds(c * 16, 16)]
        pltpu.async_copy(os_hbm.at[p16], rbuf, semA).wait()
        gvc = gvv[pl.ds(c * 16, 16)]
        for r in range(8):
            g0 = _splat(gvc, 2 * r)
            g1 = _splat(gvc, 2 * r + 1)

            def col(jo, _):
                for ji in range(8):
                    sl = pl.ds(jo * 128 + ji * 16, 16)
                    obuf[r, sl] = rbuf[2 * r, sl] * g0 + rbuf[2 * r + 1, sl] * g1
                return 0

            lax.fori_loop(0, D // 128, col, 0)
        pltpu.sync_copy(obuf, out_hbm.at[pl.ds(pl.multiple_of(base_t + c * 8, 8), 8)])
        return 0

    lax.fori_loop(0, CH // 16, c_body, 0)


def kernel(x, expert_sel, keys_w, values_w):
    B, S, D = x.shape
    E = expert_sel.shape[1]
    F = keys_w.shape[2]
    T = B * S
    NP = 2 * T
    NSLOT = NP + E * BM
    NB = NSLOT // BM
    tokens = x.reshape(T, D)

    Bt = 512
    nbt = T // Bt
    idxg, gv, hist = pl.pallas_call(
        _router_body,
        grid=(nbt,),
        in_specs=[
            pl.BlockSpec((Bt, D), lambda b: (b, 0)),
            pl.BlockSpec((D, E), lambda b: (0, 0)),
        ],
        out_specs=[
            pl.BlockSpec((Bt, 2), lambda b: (b, 0)),
            pl.BlockSpec((Bt, 2), lambda b: (b, 0)),
            pl.BlockSpec((1, 2, E), lambda b: (b, 0, 0)),
        ],
        out_shape=[
            jax.ShapeDtypeStruct((T, 2), jnp.int32),
            jax.ShapeDtypeStruct((T, 2), jnp.float32),
            jax.ShapeDtypeStruct((nbt, 2, E), jnp.int32),
        ],
    )(tokens, expert_sel)

    mesh = plsc.VectorSubcoreMesh(core_axis_name="c", subcore_axis_name="s")
    dispatch = functools.partial(
        pl.kernel,
        mesh=mesh,
        out_type=[
            jax.ShapeDtypeStruct((NSLOT, D), jnp.float32),
            jax.ShapeDtypeStruct((NP,), jnp.int32),
            jax.ShapeDtypeStruct((NB,), jnp.int32),
        ],
        scratch_types=[
            pltpu.VMEM((NP // NW,), jnp.int32),
            pltpu.VMEM((NW * E,), jnp.int32),
            pltpu.VMEM((NP // NW,), jnp.int32),
            pltpu.VMEM((RG,), jnp.int32),
            pltpu.VMEM((RG,), jnp.int32),
            pltpu.VMEM((NP // NW,), jnp.int32),
            pltpu.VMEM((NB,), jnp.int32),
            pltpu.VMEM((RG, D), jnp.float32),
            pltpu.SemaphoreType.DMA,
            pltpu.SemaphoreType.DMA,
        ],
    )(_dispatch_body)
    xs, pos, be = dispatch(idxg.reshape(NP), tokens, hist.reshape(NW * E))

    grid_spec = pltpu.PrefetchScalarGridSpec(
        num_scalar_prefetch=1,
        grid=(NB,),
        in_specs=[
            pl.BlockSpec((BM, D), lambda b, be: (b, 0)),
            pl.BlockSpec((1, D, F), lambda b, be: (jnp.maximum(be[b], 0), 0, 0)),
            pl.BlockSpec((1, F, D), lambda b, be: (jnp.maximum(be[b], 0), 0, 0)),
        ],
        out_specs=pl.BlockSpec((BM, D), lambda b, be: (b, 0)),
    )
    os_rows = pl.pallas_call(
        _gmm_body,
        grid_spec=grid_spec,
        out_shape=jax.ShapeDtypeStruct((NSLOT, D), jnp.float32),
    )(be, xs, keys_w.astype(jnp.bfloat16), values_w.astype(jnp.bfloat16))

    combine = functools.partial(
        pl.kernel,
        mesh=mesh,
        out_type=jax.ShapeDtypeStruct((T, D), jnp.float32),
        scratch_types=[
            pltpu.VMEM((NP // NW,), jnp.int32),
            pltpu.VMEM((NP // NW,), jnp.float32),
            pltpu.VMEM((16,), jnp.int32),
            pltpu.VMEM((16, D), jnp.float32),
            pltpu.VMEM((8, D), jnp.float32),
            pltpu.SemaphoreType.DMA,
        ],
    )(_combine_body)
    out = combine(os_rows, pos, gv.reshape(NP))

    return out.reshape(B, S, D)


# speedup vs baseline: 95.6204x; 1.1231x over previous
import functools

import jax
import jax.numpy as jnp
from jax import lax
from jax.experimental import pallas as pl
from jax.experimental.pallas import tpu as pltpu
from jax.experimental.pallas import tpu_sc as plsc

BM = 256
NW = 32
RG = 16


def _take16(v, idx):
    dnums = lax.GatherDimensionNumbers(
        offset_dims=(), collapsed_slice_dims=(0,), start_index_map=(0,))
    return lax.gather(v, idx[:, None], dnums, slice_sizes=(1,),
                      mode=lax.GatherScatterMode.PROMISE_IN_BOUNDS)


def _splat(v, lane):
    return _take16(v, jnp.full((16,), lane, jnp.int32))


def _prefix_incl(s, iota16):
    for k in (1, 2, 4, 8):
        g = _take16(s, jnp.maximum(iota16 - k, 0))
        s = s + jnp.where(iota16 >= k, g, jnp.zeros_like(s))
    return s


def _router_body(x_ref, sel_w_ref, idx_ref, gv_ref, hist_ref):
    Bt = x_ref.shape[0]
    E = sel_w_ref.shape[1]
    logits = jnp.dot(x_ref[...], sel_w_ref[...],
                     preferred_element_type=jnp.float32)
    s = jax.nn.sigmoid(logits)
    iota_e = lax.broadcasted_iota(jnp.int32, (Bt, E), 1)
    m1 = jnp.max(s, axis=1, keepdims=True)
    i1 = jnp.min(jnp.where(s == m1, iota_e, E), axis=1, keepdims=True)
    s2 = jnp.where(iota_e == i1, -jnp.inf, s)
    m2 = jnp.max(s2, axis=1, keepdims=True)
    i2 = jnp.min(jnp.where(s2 == m2, iota_e, E), axis=1, keepdims=True)
    idx_ref[...] = jnp.concatenate([i1, i2], axis=1)
    gv_ref[...] = jnp.concatenate([m1, m2], axis=1)
    half = Bt // 2
    rows = []
    for h in range(2):
        i1h = i1[h * half:(h + 1) * half]
        i2h = i2[h * half:(h + 1) * half]
        ioh = lax.broadcasted_iota(jnp.int32, (half, E), 1)
        cnt = ((i1h == ioh).astype(jnp.int32)
               + (i2h == ioh).astype(jnp.int32))
        rows.append(jnp.sum(cnt, axis=0, keepdims=True))
    hist_ref[...] = jnp.concatenate(rows, axis=0)[None]


def _dispatch_body(idx_hbm, x_hbm, hist_hbm, xs_hbm, pos_hbm, be_hbm,
                   idxv, histv, destv, tokA, tokB, destA, destB, tokv, bev,
                   bufA, bufB, gsemA, gsemB, ssemA, ssemB):
    T, D = x_hbm.shape
    NP = idx_hbm.shape[0]
    E = 16
    CH = NP // NW
    NB = be_hbm.shape[0]
    wid = lax.axis_index("s") * 2 + lax.axis_index("c")
    base_p = pl.multiple_of(wid * CH, 8)
    iota16 = lax.iota(jnp.int32, 16)
    ones16 = jnp.full((16,), 1, jnp.int32)
    zeros16 = jnp.zeros((16,), jnp.int32)

    pltpu.sync_copy(idx_hbm.at[pl.ds(base_p, CH)], idxv)
    pltpu.sync_copy(hist_hbm, histv)

    def h_body(c, carry):
        ctot, pre = carry
        row = histv[pl.ds(c * E, E)]
        ctot = ctot + row
        pre = pre + jnp.where(c < wid, row, zeros16)
        return ctot, pre

    ctot, pre = lax.fori_loop(0, NW, h_body, (zeros16, zeros16))
    pc = lax.shift_left(lax.shift_right_logical(ctot + (BM - 1), 8), 8)
    gs = _prefix_incl(pc, iota16) - pc
    base = gs + pre

    for q in range(NB // 16):
        bstart = (iota16 + q * 16) * BM
        be = jnp.full((16,), -1, jnp.int32)
        for e in range(E):
            gse = _splat(gs, e)
            pce = _splat(pc, e)
            m = (bstart >= gse) & (bstart < gse + pce)
            be = jnp.where(m, e, be)
        bev[pl.ds(q * 16, 16)] = be

    @pl.when(wid == 0)
    def _():
        pltpu.sync_copy(bev, be_hbm)

    def c_body(j, cnt):
        v = idxv[pl.ds(j * 16, 16)]
        dest = zeros16
        for e in range(E):
            m = v == e
            mi = jnp.where(m, ones16, zeros16)
            pref = _prefix_incl(mi, iota16)
            ce = _splat(cnt, e)
            nm = _splat(pref, 15)
            dest = jnp.where(m, ce + pref - 1, dest)
            cnt = cnt + jnp.where(iota16 == e, nm, zeros16)
        destv[pl.ds(j * 16, 16)] = dest
        p = base_p + j * 16 + iota16
        tokv[pl.ds(j * 16, 16)] = lax.shift_right_logical(p, 1)
        return cnt

    lax.fori_loop(0, CH // 16, c_body, base)
    pltpu.sync_copy(destv, pos_hbm.at[pl.ds(base_p, CH)])

    NCH = CH // RG

    def _gwait(buf, sem):
        pltpu.make_async_copy(x_hbm.at[pl.ds(0, RG)], buf, sem).wait()

    def _swait(buf, sem):
        pltpu.make_async_copy(buf, xs_hbm.at[pl.ds(0, RG)], sem).wait()

    tokA[pl.ds(0, RG)] = tokv[pl.ds(0, RG)]
    destA[pl.ds(0, RG)] = destv[pl.ds(0, RG)]
    pltpu.async_copy(x_hbm.at[tokA], bufA, gsemA)

    def d_body(j2, _):
        a = 2 * j2
        _gwait(bufA, gsemA)
        pltpu.async_copy(bufA, xs_hbm.at[destA], ssemA)

        @pl.when(j2 > 0)
        def _():
            _swait(bufB, ssemB)

        tokB[pl.ds(0, RG)] = tokv[pl.ds((a + 1) * RG, RG)]
        destB[pl.ds(0, RG)] = destv[pl.ds((a + 1) * RG, RG)]
        pltpu.async_copy(x_hbm.at[tokB], bufB, gsemB)
        _gwait(bufB, gsemB)
        pltpu.async_copy(bufB, xs_hbm.at[destB], ssemB)
        _swait(bufA, ssemA)

        @pl.when(j2 < NCH // 2 - 1)
        def _():
            tokA[pl.ds(0, RG)] = tokv[pl.ds((a + 2) * RG, RG)]
            destA[pl.ds(0, RG)] = destv[pl.ds((a + 2) * RG, RG)]
            pltpu.async_copy(x_hbm.at[tokA], bufA, gsemA)

        return 0

    lax.fori_loop(0, NCH // 2, d_body, 0)
    _swait(bufB, ssemB)


def _gmm_body(be_ref, xs_ref, k_ref, v_ref, os_ref):
    xb = xs_ref[...].astype(jnp.bfloat16)
    h = jnp.dot(xb, k_ref[0], preferred_element_type=jnp.float32)
    h = jnp.maximum(h, 0.0)
    o = jnp.dot(h.astype(jnp.bfloat16), v_ref[0],
                preferred_element_type=jnp.float32)
    os_ref[...] = o


def _combine_body(os_hbm, pos_hbm, gv_hbm, out_hbm,
                  posv, gvv, pA, pB, rbufA, rbufB, obufA, obufB,
                  gsemA, gsemB, wsemA, wsemB):
    D = os_hbm.shape[1]
    NP = pos_hbm.shape[0]
    CH = NP // NW
    wid = lax.axis_index("s") * 2 + lax.axis_index("c")
    base_p = pl.multiple_of(wid * CH, 8)
    base_t = pl.multiple_of(wid * (CH // 2), 8)
    NCH = CH // 16

    pltpu.sync_copy(pos_hbm.at[pl.ds(base_p, CH)], posv)
    pltpu.sync_copy(gv_hbm.at[pl.ds(base_p, CH)], gvv)

    def _gwait(buf, sem):
        pltpu.make_async_copy(os_hbm.at[pl.ds(0, 16)], buf, sem).wait()

    def _wwait(buf, sem):
        pltpu.make_async_copy(buf, out_hbm.at[pl.ds(0, 8)], sem).wait()

    def _emit(c, rbuf, obuf, wsem):
        gvc = gvv[pl.ds(c * 16, 16)]
        for r in range(8):
            g0 = _splat(gvc, 2 * r)
            g1 = _splat(gvc, 2 * r + 1)

            def col(jo, _):
                for ji in range(8):
                    sl = pl.ds(jo * 128 + ji * 16, 16)
                    obuf[r, sl] = rbuf[2 * r, sl] * g0 + rbuf[2 * r + 1, sl] * g1
                return 0

            lax.fori_loop(0, D // 128, col, 0)
        dst = out_hbm.at[pl.ds(pl.multiple_of(base_t + c * 8, 8), 8)]
        pltpu.async_copy(obuf, dst, wsem)

    pA[pl.ds(0, 16)] = posv[pl.ds(0, 16)]
    pltpu.async_copy(os_hbm.at[pA], rbufA, gsemA)

    def c_body(c2, _):
        a = 2 * c2
        _gwait(rbufA, gsemA)
        pB[pl.ds(0, 16)] = posv[pl.ds((a + 1) * 16, 16)]
        pltpu.async_copy(os_hbm.at[pB], rbufB, gsemB)

        @pl.when(c2 > 0)
        def _():
            _wwait(obufA, wsemA)

        _emit(a, rbufA, obufA, wsemA)
        _gwait(rbufB, gsemB)

        @pl.when(c2 > 0)
        def _():
            _wwait(obufB, wsemB)

        @pl.when(c2 < NCH // 2 - 1)
        def _():
            pA[pl.ds(0, 16)] = posv[pl.ds((a + 2) * 16, 16)]
            pltpu.async_copy(os_hbm.at[pA], rbufA, gsemA)

        _emit(a + 1, rbufB, obufB, wsemB)
        return 0

    lax.fori_loop(0, NCH // 2, c_body, 0)
    _wwait(obufA, wsemA)
    _wwait(obufB, wsemB)


def kernel(x, expert_sel, keys_w, values_w):
    B, S, D = x.shape
    E = expert_sel.shape[1]
    F = keys_w.shape[2]
    T = B * S
    NP = 2 * T
    NSLOT = NP + E * BM
    NB = NSLOT // BM
    tokens = x.reshape(T, D)

    Bt = 512
    nbt = T // Bt
    idxg, gv, hist = pl.pallas_call(
        _router_body,
        grid=(nbt,),
        in_specs=[
            pl.BlockSpec((Bt, D), lambda b: (b, 0)),
            pl.BlockSpec((D, E), lambda b: (0, 0)),
        ],
        out_specs=[
            pl.BlockSpec((Bt, 2), lambda b: (b, 0)),
            pl.BlockSpec((Bt, 2), lambda b: (b, 0)),
            pl.BlockSpec((1, 2, E), lambda b: (b, 0, 0)),
        ],
        out_shape=[
            jax.ShapeDtypeStruct((T, 2), jnp.int32),
            jax.ShapeDtypeStruct((T, 2), jnp.float32),
            jax.ShapeDtypeStruct((nbt, 2, E), jnp.int32),
        ],
    )(tokens, expert_sel)

    mesh = plsc.VectorSubcoreMesh(core_axis_name="c", subcore_axis_name="s")
    dispatch = functools.partial(
        pl.kernel,
        mesh=mesh,
        out_type=[
            jax.ShapeDtypeStruct((NSLOT, D), jnp.float32),
            jax.ShapeDtypeStruct((NP,), jnp.int32),
            jax.ShapeDtypeStruct((NB,), jnp.int32),
        ],
        scratch_types=[
            pltpu.VMEM((NP // NW,), jnp.int32),
            pltpu.VMEM((NW * E,), jnp.int32),
            pltpu.VMEM((NP // NW,), jnp.int32),
            pltpu.VMEM((RG,), jnp.int32),
            pltpu.VMEM((RG,), jnp.int32),
            pltpu.VMEM((RG,), jnp.int32),
            pltpu.VMEM((RG,), jnp.int32),
            pltpu.VMEM((NP // NW,), jnp.int32),
            pltpu.VMEM((NB,), jnp.int32),
            pltpu.VMEM((RG, D), jnp.float32),
            pltpu.VMEM((RG, D), jnp.float32),
            pltpu.SemaphoreType.DMA,
            pltpu.SemaphoreType.DMA,
            pltpu.SemaphoreType.DMA,
            pltpu.SemaphoreType.DMA,
        ],
    )(_dispatch_body)
    xs, pos, be = dispatch(idxg.reshape(NP), tokens, hist.reshape(NW * E))

    grid_spec = pltpu.PrefetchScalarGridSpec(
        num_scalar_prefetch=1,
        grid=(NB,),
        in_specs=[
            pl.BlockSpec((BM, D), lambda b, be: (b, 0)),
            pl.BlockSpec((1, D, F), lambda b, be: (jnp.maximum(be[b], 0), 0, 0)),
            pl.BlockSpec((1, F, D), lambda b, be: (jnp.maximum(be[b], 0), 0, 0)),
        ],
        out_specs=pl.BlockSpec((BM, D), lambda b, be: (b, 0)),
    )
    os_rows = pl.pallas_call(
        _gmm_body,
        grid_spec=grid_spec,
        out_shape=jax.ShapeDtypeStruct((NSLOT, D), jnp.float32),
    )(be, xs, keys_w.astype(jnp.bfloat16), values_w.astype(jnp.bfloat16))

    combine = functools.partial(
        pl.kernel,
        mesh=mesh,
        out_type=jax.ShapeDtypeStruct((T, D), jnp.float32),
        scratch_types=[
            pltpu.VMEM((NP // NW,), jnp.int32),
            pltpu.VMEM((NP // NW,), jnp.float32),
            pltpu.VMEM((16,), jnp.int32),
            pltpu.VMEM((16,), jnp.int32),
            pltpu.VMEM((16, D), jnp.float32),
            pltpu.VMEM((16, D), jnp.float32),
            pltpu.VMEM((8, D), jnp.float32),
            pltpu.VMEM((8, D), jnp.float32),
            pltpu.SemaphoreType.DMA,
            pltpu.SemaphoreType.DMA,
            pltpu.SemaphoreType.DMA,
            pltpu.SemaphoreType.DMA,
        ],
    )(_combine_body)
    out = combine(os_rows, pos, gv.reshape(NP))

    return out.reshape(B, S, D)
